# baseline (speedup 1.0000x reference)
import functools

import jax
import jax.numpy as jnp
from jax import lax
from jax.experimental import pallas as pl
from jax.experimental.pallas import tpu as pltpu
from jax.experimental.pallas import tpu_sc as plsc

N = 10000
E = 320000
D = 128
D_OUT = 64

NS = 16
EPS = E // NS
K = 80
FULL = 249
TAIL = EPS - FULL * K
RPT = 624
RTL = N - RPT * NS

R = 2000
GRID = N // R


def _seg_body(with_counts, tab_on, e_on, tab_vo, e_vo, zeros, *rest):
    if with_counts:
        (out_on, out_vo, out_cnt,
         srcb0, dstb0, srcb1, dstb1, srcb2, dstb2,
         rows0, rows1, rows2, cntloc,
         acc, g0, g1, g2, s0, s1, s2) = rest
    else:
        (out_on, out_vo,
         srcb0, dstb0, srcb1, dstb1, srcb2, dstb2,
         rows0, rows1, rows2,
         acc, g0, g1, g2, s0, s1, s2) = rest
    c = lax.axis_index("c")
    s = lax.axis_index("s")

    pltpu.sync_copy(zeros.at[pl.ds(s * RPT, RPT)], acc.at[pl.ds(s * RPT, RPT)])

    @pl.when(s == NS - 1)
    def _():
        pltpu.sync_copy(zeros.at[pl.ds(RPT * NS, RTL)],
                        acc.at[pl.ds(RPT * NS, RTL)])

    if with_counts:
        def zbody(i, carry):
            cntloc[pl.ds(pl.multiple_of(i * 16, 16), 16)] = \
                jnp.zeros((16,), jnp.float32)
            return carry
        lax.fori_loop(0, N // 16, zbody, 0)

    plsc.subcore_barrier()

    def count_chunk(db, n):
        if not with_counts:
            return
        ones = jnp.ones((16,), jnp.float32)
        for u in range(n // 16):
            idx = db[pl.ds(u * 16, 16)]
            plsc.addupdate_scatter(cntloc, [idx], ones)

    def run(tab, eref, outref):
        base = s * EPS

        def load_idx(i, sb, db):
            off = pl.multiple_of(base + i * K, 8)
            pltpu.sync_copy(eref.at[pl.ds(off, K)], sb)
            pltpu.sync_copy(eref.at[pl.ds(E + off, K)], db)
            count_chunk(db, K)

        def start_gather(sb, rows, sem):
            pltpu.async_copy(tab.at[sb], rows, sem)

        def wait_gather(sb, rows, sem):
            pltpu.make_async_copy(tab.at[sb], rows, sem).wait()

        def start_scat(db, rows, sem):
            pltpu.async_copy(rows, acc.at[db], sem, add=True)

        def wait_scat(db, rows, sem):
            pltpu.make_async_copy(rows, acc.at[db], sem).wait()

        TRIPS = FULL // 3

        load_idx(0, srcb0, dstb0)
        start_gather(srcb0, rows0, g0)
        load_idx(1, srcb1, dstb1)

        def trip(j, carry):
            c0 = j * 3
            c1 = c0 + 1
            c2 = c0 + 2

            start_gather(srcb1, rows1, g1)

            @pl.when(j > 0)
            def _():
                wait_scat(dstb2, rows2, s2)

            load_idx(c2, srcb2, dstb2)
            wait_gather(srcb0, rows0, g0)
            start_scat(dstb0, rows0, s0)
            start_gather(srcb2, rows2, g2)
            wait_scat(dstb0, rows0, s0)

            @pl.when(j < TRIPS - 1)
            def _():
                load_idx(c0 + 3, srcb0, dstb0)
                start_gather(srcb0, rows0, g0)

            wait_gather(srcb1, rows1, g1)
            start_scat(dstb1, rows1, s1)
            wait_scat(dstb1, rows1, s1)

            @pl.when(j < TRIPS - 1)
            def _():
                load_idx(c0 + 4, srcb1, dstb1)

            wait_gather(srcb2, rows2, g2)
            start_scat(dstb2, rows2, s2)
            return carry

        lax.fori_loop(0, TRIPS, trip, 0)

        load_idx(FULL, srcb0, dstb0)
        start_gather(srcb0, rows0, g0)
        wait_gather(srcb0, rows0, g0)
        start_scat(dstb0, rows0, s0)
        wait_scat(dstb0, rows0, s0)
        wait_scat(dstb2, rows2, s2)

        plsc.subcore_barrier()
        pltpu.sync_copy(acc.at[pl.ds(s * RPT, RPT)],
                        outref.at[pl.ds(s * RPT, RPT)])

        @pl.when(s == NS - 1)
        def _():
            pltpu.sync_copy(acc.at[pl.ds(RPT * NS, RTL)],
                            outref.at[pl.ds(RPT * NS, RTL)])

        if with_counts:
            pltpu.sync_copy(cntloc, out_cnt.at[c, s])

    @pl.when(c == 0)
    def _():
        run(tab_on, e_on, out_on)

    @pl.when(c == 1)
    def _():
        run(tab_vo, e_vo, out_vo)


@functools.cache
def _seg_call(with_counts):
    out_type = [jax.ShapeDtypeStruct((N, D), jnp.float32),
                jax.ShapeDtypeStruct((N, D), jnp.float32)]
    scratch = [
        pltpu.VMEM((K,), jnp.int32),
        pltpu.VMEM((K,), jnp.int32),
        pltpu.VMEM((K,), jnp.int32),
        pltpu.VMEM((K,), jnp.int32),
        pltpu.VMEM((K,), jnp.int32),
        pltpu.VMEM((K,), jnp.int32),
        pltpu.VMEM((K, D), jnp.float32),
        pltpu.VMEM((K, D), jnp.float32),
        pltpu.VMEM((K, D), jnp.float32),
    ]
    if with_counts:
        out_type = out_type + [jax.ShapeDtypeStruct((2, NS, N), jnp.float32)]
        scratch = scratch + [pltpu.VMEM((N,), jnp.float32)]
    scratch = scratch + [
        pltpu.VMEM_SHARED((N, D), jnp.float32),
        pltpu.SemaphoreType.DMA,
        pltpu.SemaphoreType.DMA,
        pltpu.SemaphoreType.DMA,
        pltpu.SemaphoreType.DMA,
        pltpu.SemaphoreType.DMA,
        pltpu.SemaphoreType.DMA,
    ]
    return pl.kernel(
        functools.partial(_seg_body, with_counts),
        out_type=out_type,
        mesh=plsc.VectorSubcoreMesh(core_axis_name="c", subcore_axis_name="s",
                                    num_cores=2, num_subcores=NS),
        compiler_params=pltpu.CompilerParams(needs_layout_passes=False),
        scratch_types=scratch,
    )


def _matT(a, w):
    return lax.dot_general(a, w, (((1,), (1,)), ((), ())),
                           preferred_element_type=jnp.float32)


def _recips(cnt_blk):
    cnt_on = jnp.sum(cnt_blk[:, :NS], axis=1, keepdims=True)
    cnt_vo = jnp.sum(cnt_blk[:, NS:], axis=1, keepdims=True)
    return (1.0 / jnp.maximum(cnt_on, 1.0), 1.0 / jnp.maximum(cnt_vo, 1.0))


def _proj_body(x_ref, won_ref, bon_ref, wvo_ref, bvo_ref, hon_ref, hvo_ref):
    xb = x_ref[...]
    hon_ref[...] = jnp.maximum(_matT(xb, won_ref[...]) + bon_ref[...], 0.0)
    hvo_ref[...] = jnp.maximum(_matT(xb, wvo_ref[...]) + bvo_ref[...], 0.0)


def _sage_out(seg, recip, xb, wl, bl, wr):
    o = _matT(seg * recip, wl) + bl + _matT(xb, wr)
    nrm = jnp.sqrt(jnp.sum(o * o, axis=-1, keepdims=True))
    return o / jnp.maximum(nrm, 1e-12)


def _combine1_body(son_ref, svo_ref, cnt_ref, x_ref,
                   wlon_ref, blon_ref, wron_ref,
                   wlvo_ref, blvo_ref, wrvo_ref,
                   g_ref, b_ref, h_ref):
    xb = x_ref[...]
    rec_on, rec_vo = _recips(cnt_ref[...])
    o_on = _sage_out(son_ref[...], rec_on, xb,
                     wlon_ref[...], blon_ref[...], wron_ref[...])
    o_vo = _sage_out(svo_ref[...], rec_vo, xb,
                     wlvo_ref[...], blvo_ref[...], wrvo_ref[...])
    h = jnp.maximum((o_on + o_vo) * 0.5, 0.0)
    mu = jnp.mean(h, axis=-1, keepdims=True)
    var = jnp.mean((h - mu) ** 2, axis=-1, keepdims=True)
    h_ref[...] = (h - mu) / jnp.sqrt(var + 1e-5) * g_ref[...] + b_ref[...]


def _combine2_body(son_ref, svo_ref, cnt_ref, h_ref,
                   wlon_ref, blon_ref, wron_ref,
                   wlvo_ref, blvo_ref, wrvo_ref,
                   w1_ref, b1_ref, w2_ref, b2_ref, out_ref):
    hb = h_ref[...]
    rec_on, rec_vo = _recips(cnt_ref[...])
    o_on = (_matT(son_ref[...] * rec_on, wlon_ref[...]) + blon_ref[...]
            + _matT(hb, wron_ref[...]))
    o_vo = (_matT(svo_ref[...] * rec_vo, wlvo_ref[...]) + blvo_ref[...]
            + _matT(hb, wrvo_ref[...]))
    o2 = (o_on + o_vo) * 0.5
    m = jnp.maximum(_matT(o2, w1_ref[...]) + b1_ref[...], 0.0)
    m = m * (1.0 / jnp.sqrt(1.0 + 1e-5))
    out_ref[...] = _matT(m, w2_ref[...]) + b2_ref[...]


def _row_spec(width):
    return pl.BlockSpec((R, width), lambda i: (i, 0))


_cnt_spec = pl.BlockSpec((R, 2 * NS), lambda i: (i, 0))


def _full_spec(shape):
    return pl.BlockSpec(shape, lambda i: (0,) * len(shape))


def _tc_call(body, in_specs, out_widths):
    return pl.pallas_call(
        body,
        grid=(GRID,),
        in_specs=in_specs,
        out_specs=[_row_spec(w) for w in out_widths],
        out_shape=[jax.ShapeDtypeStruct((N, w), jnp.float32)
                   for w in out_widths],
    )


_proj_call = _tc_call(
    _proj_body,
    [_row_spec(D)] + [_full_spec(s) for s in [(D, D), (1, D), (D, D), (1, D)]],
    [D, D])

_combine1_call = _tc_call(
    _combine1_body,
    [_row_spec(D), _row_spec(D), _cnt_spec, _row_spec(D)] +
    [_full_spec(s) for s in
     [(D, D), (1, D), (D, D), (D, D), (1, D), (D, D), (1, D), (1, D)]],
    [D])

_combine2_call = _tc_call(
    _combine2_body,
    [_row_spec(D), _row_spec(D), _cnt_spec, _row_spec(D)] +
    [_full_spec(s) for s in
     [(D, D), (1, D), (D, D), (D, D), (1, D), (D, D),
      (D, D), (1, D), (D_OUT, D), (1, D_OUT)]],
    [D_OUT])


def kernel(x, edge_index_onset, edge_index_voice, Wp_on, bp_on, Wl1_on,
           bl1_on, Wr1_on, Wp_vo, bp_vo, Wl1_vo, bl1_vo, Wr1_vo, Wl2_on,
           bl2_on, Wr2_on, Wl2_vo, bl2_vo, Wr2_vo, ln_g, ln_b, W1, b1,
           W2, b2):
    zeros = jnp.zeros((N, D), jnp.float32)

    r2 = lambda v: v.reshape(1, -1)

    h_on, h_vo = _proj_call(x, Wp_on, r2(bp_on), Wp_vo, r2(bp_vo))
    e_on = edge_index_onset.reshape(2 * E)
    e_vo = edge_index_voice.reshape(2 * E)
    s_on, s_vo, cnt_p = _seg_call(True)(h_on, e_on, h_vo, e_vo, zeros)
    cnt = jnp.transpose(cnt_p.reshape(2 * NS, N), (1, 0))
    (h,) = _combine1_call(s_on, s_vo, cnt, x,
                          Wl1_on, r2(bl1_on), Wr1_on,
                          Wl1_vo, r2(bl1_vo), Wr1_vo,
                          r2(ln_g), r2(ln_b))
    s2_on, s2_vo = _seg_call(False)(h, e_on, h, e_vo, zeros)
    (out,) = _combine2_call(s2_on, s2_vo, cnt, h,
                            Wl2_on, r2(bl2_on), Wr2_on,
                            Wl2_vo, r2(bl2_vo), Wr2_vo,
                            W1, r2(b1), W2, r2(b2))
    return out


# speedup vs baseline: 1.2194x; 1.2194x over previous
import functools

import jax
import jax.numpy as jnp
from jax import lax
from jax.experimental import pallas as pl
from jax.experimental.pallas import tpu as pltpu
from jax.experimental.pallas import tpu_sc as plsc

N = 10000
E = 320000
D = 128
D_OUT = 64

NS = 16
EPS = E // NS
K = 128
FULL = EPS // K
TAIL = EPS - FULL * K
RPT = 624
RTL = N - RPT * NS

R = 2000
GRID = N // R


def _seg_body(with_counts, tab_on, e_on, tab_vo, e_vo, zeros, *rest):
    if with_counts:
        (out_on, out_vo, out_cnt,
         srcb0, dstb0, srcb1, dstb1, rows0, rows1, src_tb, dst_tb, cntloc,
         acc, g0, g1, s0, s1) = rest
    else:
        (out_on, out_vo,
         srcb0, dstb0, srcb1, dstb1, rows0, rows1, src_tb, dst_tb,
         acc, g0, g1, s0, s1) = rest
    c = lax.axis_index("c")
    s = lax.axis_index("s")

    pltpu.sync_copy(zeros.at[pl.ds(s * RPT, RPT)], acc.at[pl.ds(s * RPT, RPT)])

    @pl.when(s == NS - 1)
    def _():
        pltpu.sync_copy(zeros.at[pl.ds(RPT * NS, RTL)],
                        acc.at[pl.ds(RPT * NS, RTL)])

    if with_counts:
        def zbody(i, carry):
            cntloc[pl.ds(pl.multiple_of(i * 16, 16), 16)] = \
                jnp.zeros((16,), jnp.float32)
            return carry
        lax.fori_loop(0, N // 16, zbody, 0)

    plsc.subcore_barrier()

    def count_chunk(db, n):
        if not with_counts:
            return
        ones = jnp.ones((16,), jnp.float32)
        for u in range(n // 16):
            idx = db[pl.ds(u * 16, 16)]
            plsc.addupdate_scatter(cntloc, [idx], ones)

    def run(tab, eref, outref):
        base = s * EPS

        def load_idx(i, sb, db):
            off = pl.multiple_of(base + i * K, 8)
            pltpu.sync_copy(eref.at[pl.ds(off, K)], sb)
            pltpu.sync_copy(eref.at[pl.ds(E + off, K)], db)
            count_chunk(db, K)

        def start_gather(sb, rows, sem):
            pltpu.async_copy(tab.at[sb], rows, sem)

        def wait_gather(sb, rows, sem):
            pltpu.make_async_copy(tab.at[sb], rows, sem).wait()

        def start_scat(db, rows, sem):
            pltpu.async_copy(rows, acc.at[db], sem, add=True)

        def wait_scat(db, rows, sem):
            pltpu.make_async_copy(rows, acc.at[db], sem).wait()

        PAIRS = FULL // 2

        load_idx(0, srcb0, dstb0)
        start_gather(srcb0, rows0, g0)

        def pair(j, carry):
            c0 = j * 2
            c1 = c0 + 1

            @pl.when(j > 0)
            def _():
                wait_scat(dstb1, rows1, s1)

            load_idx(c1, srcb1, dstb1)
            start_gather(srcb1, rows1, g1)

            wait_gather(srcb0, rows0, g0)
            start_scat(dstb0, rows0, s0)
            wait_scat(dstb0, rows0, s0)

            @pl.when(j < PAIRS - 1)
            def _():
                load_idx(c0 + 2, srcb0, dstb0)
                start_gather(srcb0, rows0, g0)

            wait_gather(srcb1, rows1, g1)
            start_scat(dstb1, rows1, s1)
            return carry

        lax.fori_loop(0, PAIRS, pair, 0)

        toff = pl.multiple_of(base + FULL * K, 8)
        pltpu.sync_copy(eref.at[pl.ds(toff, TAIL)], src_tb)
        pltpu.sync_copy(eref.at[pl.ds(E + toff, TAIL)], dst_tb)
        count_chunk(dst_tb, TAIL)
        rows_t = rows0.at[pl.ds(0, TAIL)]
        pltpu.async_copy(tab.at[src_tb], rows_t, g0).wait()
        pltpu.async_copy(rows_t, acc.at[dst_tb], s0, add=True).wait()
        wait_scat(dstb1, rows1, s1)

        plsc.subcore_barrier()
        pltpu.sync_copy(acc.at[pl.ds(s * RPT, RPT)],
                        outref.at[pl.ds(s * RPT, RPT)])

        @pl.when(s == NS - 1)
        def _():
            pltpu.sync_copy(acc.at[pl.ds(RPT * NS, RTL)],
                            outref.at[pl.ds(RPT * NS, RTL)])

        if with_counts:
            pltpu.sync_copy(cntloc, out_cnt.at[c, s])

    @pl.when(c == 0)
    def _():
        run(tab_on, e_on, out_on)

    @pl.when(c == 1)
    def _():
        run(tab_vo, e_vo, out_vo)


@functools.cache
def _seg_call(with_counts):
    out_type = [jax.ShapeDtypeStruct((N, D), jnp.float32),
                jax.ShapeDtypeStruct((N, D), jnp.float32)]
    scratch = [
        pltpu.VMEM((K,), jnp.int32),
        pltpu.VMEM((K,), jnp.int32),
        pltpu.VMEM((K,), jnp.int32),
        pltpu.VMEM((K,), jnp.int32),
        pltpu.VMEM((K, D), jnp.float32),
        pltpu.VMEM((K, D), jnp.float32),
        pltpu.VMEM((TAIL,), jnp.int32),
        pltpu.VMEM((TAIL,), jnp.int32),
    ]
    if with_counts:
        out_type = out_type + [jax.ShapeDtypeStruct((2, NS, N), jnp.float32)]
        scratch = scratch + [pltpu.VMEM((N,), jnp.float32)]
    scratch = scratch + [
        pltpu.VMEM_SHARED((N, D), jnp.float32),
        pltpu.SemaphoreType.DMA,
        pltpu.SemaphoreType.DMA,
        pltpu.SemaphoreType.DMA,
        pltpu.SemaphoreType.DMA,
    ]
    return pl.kernel(
        functools.partial(_seg_body, with_counts),
        out_type=out_type,
        mesh=plsc.VectorSubcoreMesh(core_axis_name="c", subcore_axis_name="s",
                                    num_cores=2, num_subcores=NS),
        compiler_params=pltpu.CompilerParams(needs_layout_passes=False),
        scratch_types=scratch,
    )


def _matT(a, w):
    return lax.dot_general(a, w, (((1,), (1,)), ((), ())),
                           preferred_element_type=jnp.float32)


def _recips(cnt_blk):
    cnt_on = jnp.sum(cnt_blk[:, :NS], axis=1, keepdims=True)
    cnt_vo = jnp.sum(cnt_blk[:, NS:], axis=1, keepdims=True)
    return (1.0 / jnp.maximum(cnt_on, 1.0), 1.0 / jnp.maximum(cnt_vo, 1.0))


def _proj_body(x_ref, won_ref, bon_ref, wvo_ref, bvo_ref, hon_ref, hvo_ref):
    xb = x_ref[...]
    hon_ref[...] = jnp.maximum(_matT(xb, won_ref[...]) + bon_ref[...], 0.0)
    hvo_ref[...] = jnp.maximum(_matT(xb, wvo_ref[...]) + bvo_ref[...], 0.0)


def _sage_out(seg, recip, xb, wl, bl, wr):
    o = _matT(seg * recip, wl) + bl + _matT(xb, wr)
    nrm = jnp.sqrt(jnp.sum(o * o, axis=-1, keepdims=True))
    return o / jnp.maximum(nrm, 1e-12)


def _combine1_body(son_ref, svo_ref, cnt_ref, x_ref,
                   wlon_ref, blon_ref, wron_ref,
                   wlvo_ref, blvo_ref, wrvo_ref,
                   g_ref, b_ref, h_ref):
    xb = x_ref[...]
    rec_on, rec_vo = _recips(cnt_ref[...])
    o_on = _sage_out(son_ref[...], rec_on, xb,
                     wlon_ref[...], blon_ref[...], wron_ref[...])
    o_vo = _sage_out(svo_ref[...], rec_vo, xb,
                     wlvo_ref[...], blvo_ref[...], wrvo_ref[...])
    h = jnp.maximum((o_on + o_vo) * 0.5, 0.0)
    mu = jnp.mean(h, axis=-1, keepdims=True)
    var = jnp.mean((h - mu) ** 2, axis=-1, keepdims=True)
    h_ref[...] = (h - mu) / jnp.sqrt(var + 1e-5) * g_ref[...] + b_ref[...]


def _combine2_body(son_ref, svo_ref, cnt_ref, h_ref,
                   wlon_ref, blon_ref, wron_ref,
                   wlvo_ref, blvo_ref, wrvo_ref,
                   w1_ref, b1_ref, w2_ref, b2_ref, out_ref):
    hb = h_ref[...]
    rec_on, rec_vo = _recips(cnt_ref[...])
    o_on = (_matT(son_ref[...] * rec_on, wlon_ref[...]) + blon_ref[...]
            + _matT(hb, wron_ref[...]))
    o_vo = (_matT(svo_ref[...] * rec_vo, wlvo_ref[...]) + blvo_ref[...]
            + _matT(hb, wrvo_ref[...]))
    o2 = (o_on + o_vo) * 0.5
    m = jnp.maximum(_matT(o2, w1_ref[...]) + b1_ref[...], 0.0)
    m = m * (1.0 / jnp.sqrt(1.0 + 1e-5))
    out_ref[...] = _matT(m, w2_ref[...]) + b2_ref[...]


def _row_spec(width):
    return pl.BlockSpec((R, width), lambda i: (i, 0))


_cnt_spec = pl.BlockSpec((R, 2 * NS), lambda i: (i, 0))


def _full_spec(shape):
    return pl.BlockSpec(shape, lambda i: (0,) * len(shape))


def _tc_call(body, in_specs, out_widths):
    return pl.pallas_call(
        body,
        grid=(GRID,),
        in_specs=in_specs,
        out_specs=[_row_spec(w) for w in out_widths],
        out_shape=[jax.ShapeDtypeStruct((N, w), jnp.float32)
                   for w in out_widths],
    )


_proj_call = _tc_call(
    _proj_body,
    [_row_spec(D)] + [_full_spec(s) for s in [(D, D), (1, D), (D, D), (1, D)]],
    [D, D])

_combine1_call = _tc_call(
    _combine1_body,
    [_row_spec(D), _row_spec(D), _cnt_spec, _row_spec(D)] +
    [_full_spec(s) for s in
     [(D, D), (1, D), (D, D), (D, D), (1, D), (D, D), (1, D), (1, D)]],
    [D])

_combine2_call = _tc_call(
    _combine2_body,
    [_row_spec(D), _row_spec(D), _cnt_spec, _row_spec(D)] +
    [_full_spec(s) for s in
     [(D, D), (1, D), (D, D), (D, D), (1, D), (D, D),
      (D, D), (1, D), (D_OUT, D), (1, D_OUT)]],
    [D_OUT])


def kernel(x, edge_index_onset, edge_index_voice, Wp_on, bp_on, Wl1_on,
           bl1_on, Wr1_on, Wp_vo, bp_vo, Wl1_vo, bl1_vo, Wr1_vo, Wl2_on,
           bl2_on, Wr2_on, Wl2_vo, bl2_vo, Wr2_vo, ln_g, ln_b, W1, b1,
           W2, b2):
    zeros = jnp.zeros((N, D), jnp.float32)

    r2 = lambda v: v.reshape(1, -1)

    h_on, h_vo = _proj_call(x, Wp_on, r2(bp_on), Wp_vo, r2(bp_vo))
    e_on = edge_index_onset.reshape(2 * E)
    e_vo = edge_index_voice.reshape(2 * E)
    s_on, s_vo, cnt_p = _seg_call(True)(h_on, e_on, h_vo, e_vo, zeros)
    cnt = jnp.transpose(cnt_p.reshape(2 * NS, N), (1, 0))
    (h,) = _combine1_call(s_on, s_vo, cnt, x,
                          Wl1_on, r2(bl1_on), Wr1_on,
                          Wl1_vo, r2(bl1_vo), Wr1_vo,
                          r2(ln_g), r2(ln_b))
    s2_on, s2_vo = _seg_call(False)(h, e_on, h, e_vo, zeros)
    (out,) = _combine2_call(s2_on, s2_vo, cnt, h,
                            Wl2_on, r2(bl2_on), Wr2_on,
                            Wl2_vo, r2(bl2_vo), Wr2_vo,
                            W1, r2(b1), W2, r2(b2))
    return out


# speedup vs baseline: 1.4576x; 1.1953x over previous
import functools

import jax
import jax.numpy as jnp
from jax import lax
from jax.experimental import pallas as pl
from jax.experimental.pallas import tpu as pltpu
from jax.experimental.pallas import tpu_sc as plsc

N = 10000
E = 320000
D = 128
D_OUT = 64

NS = 16
EPS = E // NS
K = 128
FULL = EPS // K
TAIL = EPS - FULL * K
RPT = 624
RTL = N - RPT * NS

R = 2000
GRID = N // R


def _seg_body(with_counts, tab_on, e_on, tab_vo, e_vo, zeros, *rest):
    if with_counts:
        (out_on, out_vo, out_cnt,
         srcb0, dstb0, srcb1, dstb1, rows0, rows1, src_tb, dst_tb, cntloc,
         acc, i0, i1, g0, g1, s0, s1) = rest
    else:
        (out_on, out_vo,
         srcb0, dstb0, srcb1, dstb1, rows0, rows1, src_tb, dst_tb,
         acc, i0, i1, g0, g1, s0, s1) = rest
    c = lax.axis_index("c")
    s = lax.axis_index("s")

    pltpu.sync_copy(zeros.at[pl.ds(s * RPT, RPT)], acc.at[pl.ds(s * RPT, RPT)])

    @pl.when(s == NS - 1)
    def _():
        pltpu.sync_copy(zeros.at[pl.ds(RPT * NS, RTL)],
                        acc.at[pl.ds(RPT * NS, RTL)])

    if with_counts:
        def zbody(i, carry):
            cntloc[pl.ds(pl.multiple_of(i * 16, 16), 16)] = \
                jnp.zeros((16,), jnp.float32)
            return carry
        lax.fori_loop(0, N // 16, zbody, 0)

    plsc.subcore_barrier()

    def count_chunk(db, n):
        if not with_counts:
            return
        ones = jnp.ones((16,), jnp.float32)
        for u in range(n // 16):
            idx = db[pl.ds(u * 16, 16)]
            plsc.addupdate_scatter(cntloc, [idx], ones)

    def run(tab, eref, outref):
        base = s * EPS

        def load_idx(i, sb, db, sem):
            off = pl.multiple_of(base + i * K, 8)
            pltpu.async_copy(eref.at[pl.ds(off, K)], sb, sem)
            pltpu.async_copy(eref.at[pl.ds(E + off, K)], db, sem)
            pltpu.make_async_copy(eref.at[pl.ds(off, K)], sb, sem).wait()
            pltpu.make_async_copy(eref.at[pl.ds(E + off, K)], db, sem).wait()
            count_chunk(db, K)

        def start_gather(sb, rows, sem):
            pltpu.async_copy(tab.at[sb], rows, sem)

        def wait_gather(sb, rows, sem):
            pltpu.make_async_copy(tab.at[sb], rows, sem).wait()

        def start_scat(db, rows, sem):
            pltpu.async_copy(rows, acc.at[db], sem, add=True)

        def wait_scat(db, rows, sem):
            pltpu.make_async_copy(rows, acc.at[db], sem).wait()

        PAIRS = FULL // 2

        load_idx(0, srcb0, dstb0, i0)
        start_gather(srcb0, rows0, g0)

        def pair(j, carry):
            c0 = j * 2
            c1 = c0 + 1

            @pl.when(j > 0)
            def _():
                wait_scat(dstb1, rows1, s1)

            load_idx(c1, srcb1, dstb1, i1)
            start_gather(srcb1, rows1, g1)

            wait_gather(srcb0, rows0, g0)
            start_scat(dstb0, rows0, s0)
            wait_scat(dstb0, rows0, s0)

            @pl.when(j < PAIRS - 1)
            def _():
                load_idx(c0 + 2, srcb0, dstb0, i0)
                start_gather(srcb0, rows0, g0)

            wait_gather(srcb1, rows1, g1)
            start_scat(dstb1, rows1, s1)
            return carry

        lax.fori_loop(0, PAIRS, pair, 0)

        toff = pl.multiple_of(base + FULL * K, 8)
        pltpu.sync_copy(eref.at[pl.ds(toff, TAIL)], src_tb)
        pltpu.sync_copy(eref.at[pl.ds(E + toff, TAIL)], dst_tb)
        count_chunk(dst_tb, TAIL)
        rows_t = rows0.at[pl.ds(0, TAIL)]
        pltpu.async_copy(tab.at[src_tb], rows_t, g0).wait()
        pltpu.async_copy(rows_t, acc.at[dst_tb], s0, add=True).wait()
        wait_scat(dstb1, rows1, s1)

        plsc.subcore_barrier()
        pltpu.sync_copy(acc.at[pl.ds(s * RPT, RPT)],
                        outref.at[pl.ds(s * RPT, RPT)])

        @pl.when(s == NS - 1)
        def _():
            pltpu.sync_copy(acc.at[pl.ds(RPT * NS, RTL)],
                            outref.at[pl.ds(RPT * NS, RTL)])

        if with_counts:
            pltpu.sync_copy(cntloc, out_cnt.at[c, s])

    @pl.when(c == 0)
    def _():
        run(tab_on, e_on, out_on)

    @pl.when(c == 1)
    def _():
        run(tab_vo, e_vo, out_vo)


@functools.cache
def _seg_call(with_counts):
    out_type = [jax.ShapeDtypeStruct((N, D), jnp.float32),
                jax.ShapeDtypeStruct((N, D), jnp.float32)]
    scratch = [
        pltpu.VMEM((K,), jnp.int32),
        pltpu.VMEM((K,), jnp.int32),
        pltpu.VMEM((K,), jnp.int32),
        pltpu.VMEM((K,), jnp.int32),
        pltpu.VMEM((K, D), jnp.float32),
        pltpu.VMEM((K, D), jnp.float32),
        pltpu.VMEM((TAIL,), jnp.int32),
        pltpu.VMEM((TAIL,), jnp.int32),
    ]
    if with_counts:
        out_type = out_type + [jax.ShapeDtypeStruct((2, NS, N), jnp.float32)]
        scratch = scratch + [pltpu.VMEM((N,), jnp.float32)]
    scratch = scratch + [
        pltpu.VMEM_SHARED((N, D), jnp.float32),
        pltpu.SemaphoreType.DMA,
        pltpu.SemaphoreType.DMA,
        pltpu.SemaphoreType.DMA,
        pltpu.SemaphoreType.DMA,
        pltpu.SemaphoreType.DMA,
        pltpu.SemaphoreType.DMA,
    ]
    return pl.kernel(
        functools.partial(_seg_body, with_counts),
        out_type=out_type,
        mesh=plsc.VectorSubcoreMesh(core_axis_name="c", subcore_axis_name="s",
                                    num_cores=2, num_subcores=NS),
        compiler_params=pltpu.CompilerParams(needs_layout_passes=False),
        scratch_types=scratch,
    )


def _matT(a, w):
    return lax.dot_general(a, w, (((1,), (1,)), ((), ())),
                           preferred_element_type=jnp.float32)


def _recips(cnt_blk):
    cnt_on = jnp.sum(cnt_blk[:, :NS], axis=1, keepdims=True)
    cnt_vo = jnp.sum(cnt_blk[:, NS:], axis=1, keepdims=True)
    return (1.0 / jnp.maximum(cnt_on, 1.0), 1.0 / jnp.maximum(cnt_vo, 1.0))


def _proj_body(x_ref, won_ref, bon_ref, wvo_ref, bvo_ref, hon_ref, hvo_ref):
    xb = x_ref[...]
    hon_ref[...] = jnp.maximum(_matT(xb, won_ref[...]) + bon_ref[...], 0.0)
    hvo_ref[...] = jnp.maximum(_matT(xb, wvo_ref[...]) + bvo_ref[...], 0.0)


def _sage_out(seg, recip, xb, wl, bl, wr):
    o = _matT(seg * recip, wl) + bl + _matT(xb, wr)
    nrm = jnp.sqrt(jnp.sum(o * o, axis=-1, keepdims=True))
    return o / jnp.maximum(nrm, 1e-12)


def _combine1_body(son_ref, svo_ref, cnt_ref, x_ref,
                   wlon_ref, blon_ref, wron_ref,
                   wlvo_ref, blvo_ref, wrvo_ref,
                   g_ref, b_ref, h_ref):
    xb = x_ref[...]
    rec_on, rec_vo = _recips(cnt_ref[...])
    o_on = _sage_out(son_ref[...], rec_on, xb,
                     wlon_ref[...], blon_ref[...], wron_ref[...])
    o_vo = _sage_out(svo_ref[...], rec_vo, xb,
                     wlvo_ref[...], blvo_ref[...], wrvo_ref[...])
    h = jnp.maximum((o_on + o_vo) * 0.5, 0.0)
    mu = jnp.mean(h, axis=-1, keepdims=True)
    var = jnp.mean((h - mu) ** 2, axis=-1, keepdims=True)
    h_ref[...] = (h - mu) / jnp.sqrt(var + 1e-5) * g_ref[...] + b_ref[...]


def _combine2_body(son_ref, svo_ref, cnt_ref, h_ref,
                   wlon_ref, blon_ref, wron_ref,
                   wlvo_ref, blvo_ref, wrvo_ref,
                   w1_ref, b1_ref, w2_ref, b2_ref, out_ref):
    hb = h_ref[...]
    rec_on, rec_vo = _recips(cnt_ref[...])
    o_on = (_matT(son_ref[...] * rec_on, wlon_ref[...]) + blon_ref[...]
            + _matT(hb, wron_ref[...]))
    o_vo = (_matT(svo_ref[...] * rec_vo, wlvo_ref[...]) + blvo_ref[...]
            + _matT(hb, wrvo_ref[...]))
    o2 = (o_on + o_vo) * 0.5
    m = jnp.maximum(_matT(o2, w1_ref[...]) + b1_ref[...], 0.0)
    m = m * (1.0 / jnp.sqrt(1.0 + 1e-5))
    out_ref[...] = _matT(m, w2_ref[...]) + b2_ref[...]


def _row_spec(width):
    return pl.BlockSpec((R, width), lambda i: (i, 0))


_cnt_spec = pl.BlockSpec((R, 2 * NS), lambda i: (i, 0))


def _full_spec(shape):
    return pl.BlockSpec(shape, lambda i: (0,) * len(shape))


def _tc_call(body, in_specs, out_widths):
    return pl.pallas_call(
        body,
        grid=(GRID,),
        in_specs=in_specs,
        out_specs=[_row_spec(w) for w in out_widths],
        out_shape=[jax.ShapeDtypeStruct((N, w), jnp.float32)
                   for w in out_widths],
    )


_proj_call = _tc_call(
    _proj_body,
    [_row_spec(D)] + [_full_spec(s) for s in [(D, D), (1, D), (D, D), (1, D)]],
    [D, D])

_combine1_call = _tc_call(
    _combine1_body,
    [_row_spec(D), _row_spec(D), _cnt_spec, _row_spec(D)] +
    [_full_spec(s) for s in
     [(D, D), (1, D), (D, D), (D, D), (1, D), (D, D), (1, D), (1, D)]],
    [D])

_combine2_call = _tc_call(
    _combine2_body,
    [_row_spec(D), _row_spec(D), _cnt_spec, _row_spec(D)] +
    [_full_spec(s) for s in
     [(D, D), (1, D), (D, D), (D, D), (1, D), (D, D),
      (D, D), (1, D), (D_OUT, D), (1, D_OUT)]],
    [D_OUT])


def kernel(x, edge_index_onset, edge_index_voice, Wp_on, bp_on, Wl1_on,
           bl1_on, Wr1_on, Wp_vo, bp_vo, Wl1_vo, bl1_vo, Wr1_vo, Wl2_on,
           bl2_on, Wr2_on, Wl2_vo, bl2_vo, Wr2_vo, ln_g, ln_b, W1, b1,
           W2, b2):
    zeros = jnp.zeros((N, D), jnp.float32)

    r2 = lambda v: v.reshape(1, -1)

    h_on, h_vo = _proj_call(x, Wp_on, r2(bp_on), Wp_vo, r2(bp_vo))
    e_on = edge_index_onset.reshape(2 * E)
    e_vo = edge_index_voice.reshape(2 * E)
    s_on, s_vo, cnt_p = _seg_call(True)(h_on, e_on, h_vo, e_vo, zeros)
    cnt = jnp.transpose(cnt_p.reshape(2 * NS, N), (1, 0))
    (h,) = _combine1_call(s_on, s_vo, cnt, x,
                          Wl1_on, r2(bl1_on), Wr1_on,
                          Wl1_vo, r2(bl1_vo), Wr1_vo,
                          r2(ln_g), r2(ln_b))
    s2_on, s2_vo = _seg_call(False)(h, e_on, h, e_vo, zeros)
    (out,) = _combine2_call(s2_on, s2_vo, cnt, h,
                            Wl2_on, r2(bl2_on), Wr2_on,
                            Wl2_vo, r2(bl2_vo), Wr2_vo,
                            W1, r2(b1), W2, r2(b2))
    return out


# speedup vs baseline: 1.4811x; 1.0162x over previous
import functools

import jax
import jax.numpy as jnp
from jax import lax
from jax.experimental import pallas as pl
from jax.experimental.pallas import tpu as pltpu
from jax.experimental.pallas import tpu_sc as plsc

N = 10000
E = 320000
D = 128
D_OUT = 64

NS = 16
EPS = E // NS
K = 128
FULL = EPS // K
TAIL = EPS - FULL * K
RPT = 624
RTL = N - RPT * NS

R = 2000
GRID = N // R


def _seg_body(with_counts, tab_on, e_on, tab_vo, e_vo, zeros, *rest):
    nslots = 2 if with_counts else 3
    out_on, out_vo = rest[0], rest[1]
    rest = rest[2:]
    if with_counts:
        out_cnt, rest = rest[0], rest[1:]
    srcb = rest[:nslots]; rest = rest[nslots:]
    dstb = rest[:nslots]; rest = rest[nslots:]
    rows = rest[:nslots]; rest = rest[nslots:]
    src_tb, dst_tb = rest[0], rest[1]; rest = rest[2:]
    if with_counts:
        cntloc, rest = rest[0], rest[1:]
    acc = rest[0]; rest = rest[1:]
    isem = rest[:nslots]; rest = rest[nslots:]
    gsem = rest[:nslots]; rest = rest[nslots:]
    ssem = rest[:nslots]; rest = rest[nslots:]
    c = lax.axis_index("c")
    s = lax.axis_index("s")

    pltpu.sync_copy(zeros.at[pl.ds(s * RPT, RPT)], acc.at[pl.ds(s * RPT, RPT)])

    @pl.when(s == NS - 1)
    def _():
        pltpu.sync_copy(zeros.at[pl.ds(RPT * NS, RTL)],
                        acc.at[pl.ds(RPT * NS, RTL)])

    if with_counts:
        def zbody(i, carry):
            cntloc[pl.ds(pl.multiple_of(i * 16, 16), 16)] = \
                jnp.zeros((16,), jnp.float32)
            return carry
        lax.fori_loop(0, N // 16, zbody, 0)

    plsc.subcore_barrier()

    def count_chunk(db, n):
        if not with_counts:
            return
        ones = jnp.ones((16,), jnp.float32)
        for u in range(n // 16):
            idx = db[pl.ds(u * 16, 16)]
            plsc.addupdate_scatter(cntloc, [idx], ones)

    def run(tab, eref, outref):
        base = s * EPS

        def load_idx(i, sb, db, sem):
            off = pl.multiple_of(base + i * K, 8)
            pltpu.async_copy(eref.at[pl.ds(off, K)], sb, sem)
            pltpu.async_copy(eref.at[pl.ds(E + off, K)], db, sem)
            pltpu.make_async_copy(eref.at[pl.ds(off, K)], sb, sem).wait()
            pltpu.make_async_copy(eref.at[pl.ds(E + off, K)], db, sem).wait()
            count_chunk(db, K)

        def start_gather(sb, rows, sem):
            pltpu.async_copy(tab.at[sb], rows, sem)

        def wait_gather(sb, rows, sem):
            pltpu.make_async_copy(tab.at[sb], rows, sem).wait()

        def start_scat(db, rows, sem):
            pltpu.async_copy(rows, acc.at[db], sem, add=True)

        def wait_scat(db, rows, sem):
            pltpu.make_async_copy(rows, acc.at[db], sem).wait()

        if nslots == 2:
            PAIRS = FULL // 2

            load_idx(0, srcb[0], dstb[0], isem[0])
            start_gather(srcb[0], rows[0], gsem[0])

            def pair(j, carry):
                c0 = j * 2

                @pl.when(j > 0)
                def _():
                    wait_scat(dstb[1], rows[1], ssem[1])

                load_idx(c0 + 1, srcb[1], dstb[1], isem[1])
                start_gather(srcb[1], rows[1], gsem[1])

                wait_gather(srcb[0], rows[0], gsem[0])
                start_scat(dstb[0], rows[0], ssem[0])
                wait_scat(dstb[0], rows[0], ssem[0])

                @pl.when(j < PAIRS - 1)
                def _():
                    load_idx(c0 + 2, srcb[0], dstb[0], isem[0])
                    start_gather(srcb[0], rows[0], gsem[0])

                wait_gather(srcb[1], rows[1], gsem[1])
                start_scat(dstb[1], rows[1], ssem[1])
                return carry

            lax.fori_loop(0, PAIRS, pair, 0)
            last_s, last_d = ssem[1], dstb[1]
            last_r = rows[1]
        else:
            TRIPS = FULL // 3

            load_idx(0, srcb[0], dstb[0], isem[0])
            start_gather(srcb[0], rows[0], gsem[0])
            load_idx(1, srcb[1], dstb[1], isem[1])

            def trip(j, carry):
                c0 = j * 3

                start_gather(srcb[1], rows[1], gsem[1])

                @pl.when(j > 0)
                def _():
                    wait_scat(dstb[2], rows[2], ssem[2])

                load_idx(c0 + 2, srcb[2], dstb[2], isem[2])
                wait_gather(srcb[0], rows[0], gsem[0])
                start_scat(dstb[0], rows[0], ssem[0])
                start_gather(srcb[2], rows[2], gsem[2])
                wait_scat(dstb[0], rows[0], ssem[0])

                @pl.when(j < TRIPS - 1)
                def _():
                    load_idx(c0 + 3, srcb[0], dstb[0], isem[0])
                    start_gather(srcb[0], rows[0], gsem[0])

                wait_gather(srcb[1], rows[1], gsem[1])
                start_scat(dstb[1], rows[1], ssem[1])
                wait_scat(dstb[1], rows[1], ssem[1])

                @pl.when(j < TRIPS - 1)
                def _():
                    load_idx(c0 + 4, srcb[1], dstb[1], isem[1])

                wait_gather(srcb[2], rows[2], gsem[2])
                start_scat(dstb[2], rows[2], ssem[2])
                return carry

            lax.fori_loop(0, TRIPS, trip, 0)
            last_s, last_d = ssem[2], dstb[2]
            last_r = rows[2]

        toff = pl.multiple_of(base + FULL * K, 8)
        pltpu.sync_copy(eref.at[pl.ds(toff, TAIL)], src_tb)
        pltpu.sync_copy(eref.at[pl.ds(E + toff, TAIL)], dst_tb)
        count_chunk(dst_tb, TAIL)
        rows_t = rows[0].at[pl.ds(0, TAIL)]
        pltpu.async_copy(tab.at[src_tb], rows_t, gsem[0]).wait()
        pltpu.async_copy(rows_t, acc.at[dst_tb], ssem[0], add=True).wait()
        wait_scat(last_d, last_r, last_s)

        plsc.subcore_barrier()
        pltpu.sync_copy(acc.at[pl.ds(s * RPT, RPT)],
                        outref.at[pl.ds(s * RPT, RPT)])

        @pl.when(s == NS - 1)
        def _():
            pltpu.sync_copy(acc.at[pl.ds(RPT * NS, RTL)],
                            outref.at[pl.ds(RPT * NS, RTL)])

        if with_counts:
            pltpu.sync_copy(cntloc, out_cnt.at[c, s])

    @pl.when(c == 0)
    def _():
        run(tab_on, e_on, out_on)

    @pl.when(c == 1)
    def _():
        run(tab_vo, e_vo, out_vo)


@functools.cache
def _seg_call(with_counts):
    nslots = 2 if with_counts else 3
    out_type = [jax.ShapeDtypeStruct((N, D), jnp.float32),
                jax.ShapeDtypeStruct((N, D), jnp.float32)]
    if with_counts:
        out_type = out_type + [jax.ShapeDtypeStruct((2, NS, N), jnp.float32)]
    scratch = ([pltpu.VMEM((K,), jnp.int32)] * (2 * nslots) +
               [pltpu.VMEM((K, D), jnp.float32)] * nslots +
               [pltpu.VMEM((TAIL,), jnp.int32)] * 2)
    if with_counts:
        scratch = scratch + [pltpu.VMEM((N,), jnp.float32)]
    scratch = (scratch +
               [pltpu.VMEM_SHARED((N, D), jnp.float32)] +
               [pltpu.SemaphoreType.DMA] * (3 * nslots))
    return pl.kernel(
        functools.partial(_seg_body, with_counts),
        out_type=out_type,
        mesh=plsc.VectorSubcoreMesh(core_axis_name="c", subcore_axis_name="s",
                                    num_cores=2, num_subcores=NS),
        compiler_params=pltpu.CompilerParams(needs_layout_passes=False),
        scratch_types=scratch,
    )


def _matT(a, w):
    return lax.dot_general(a, w, (((1,), (1,)), ((), ())),
                           preferred_element_type=jnp.float32)


def _recips(cnt_blk):
    cnt_on = jnp.sum(cnt_blk[:, :NS], axis=1, keepdims=True)
    cnt_vo = jnp.sum(cnt_blk[:, NS:], axis=1, keepdims=True)
    return (1.0 / jnp.maximum(cnt_on, 1.0), 1.0 / jnp.maximum(cnt_vo, 1.0))


def _proj_body(x_ref, won_ref, bon_ref, wvo_ref, bvo_ref, hon_ref, hvo_ref):
    xb = x_ref[...]
    hon_ref[...] = jnp.maximum(_matT(xb, won_ref[...]) + bon_ref[...], 0.0)
    hvo_ref[...] = jnp.maximum(_matT(xb, wvo_ref[...]) + bvo_ref[...], 0.0)


def _sage_out(seg, recip, xb, wl, bl, wr):
    o = _matT(seg * recip, wl) + bl + _matT(xb, wr)
    nrm = jnp.sqrt(jnp.sum(o * o, axis=-1, keepdims=True))
    return o / jnp.maximum(nrm, 1e-12)


def _combine1_body(son_ref, svo_ref, cnt_ref, x_ref,
                   wlon_ref, blon_ref, wron_ref,
                   wlvo_ref, blvo_ref, wrvo_ref,
                   g_ref, b_ref, h_ref):
    xb = x_ref[...]
    rec_on, rec_vo = _recips(cnt_ref[...])
    o_on = _sage_out(son_ref[...], rec_on, xb,
                     wlon_ref[...], blon_ref[...], wron_ref[...])
    o_vo = _sage_out(svo_ref[...], rec_vo, xb,
                     wlvo_ref[...], blvo_ref[...], wrvo_ref[...])
    h = jnp.maximum((o_on + o_vo) * 0.5, 0.0)
    mu = jnp.mean(h, axis=-1, keepdims=True)
    var = jnp.mean((h - mu) ** 2, axis=-1, keepdims=True)
    h_ref[...] = (h - mu) / jnp.sqrt(var + 1e-5) * g_ref[...] + b_ref[...]


def _combine2_body(son_ref, svo_ref, cnt_ref, h_ref,
                   wlon_ref, blon_ref, wron_ref,
                   wlvo_ref, blvo_ref, wrvo_ref,
                   w1_ref, b1_ref, w2_ref, b2_ref, out_ref):
    hb = h_ref[...]
    rec_on, rec_vo = _recips(cnt_ref[...])
    o_on = (_matT(son_ref[...] * rec_on, wlon_ref[...]) + blon_ref[...]
            + _matT(hb, wron_ref[...]))
    o_vo = (_matT(svo_ref[...] * rec_vo, wlvo_ref[...]) + blvo_ref[...]
            + _matT(hb, wrvo_ref[...]))
    o2 = (o_on + o_vo) * 0.5
    m = jnp.maximum(_matT(o2, w1_ref[...]) + b1_ref[...], 0.0)
    m = m * (1.0 / jnp.sqrt(1.0 + 1e-5))
    out_ref[...] = _matT(m, w2_ref[...]) + b2_ref[...]


def _row_spec(width):
    return pl.BlockSpec((R, width), lambda i: (i, 0))


_cnt_spec = pl.BlockSpec((R, 2 * NS), lambda i: (i, 0))


def _full_spec(shape):
    return pl.BlockSpec(shape, lambda i: (0,) * len(shape))


def _tc_call(body, in_specs, out_widths):
    return pl.pallas_call(
        body,
        grid=(GRID,),
        in_specs=in_specs,
        out_specs=[_row_spec(w) for w in out_widths],
        out_shape=[jax.ShapeDtypeStruct((N, w), jnp.float32)
                   for w in out_widths],
    )


_proj_call = _tc_call(
    _proj_body,
    [_row_spec(D)] + [_full_spec(s) for s in [(D, D), (1, D), (D, D), (1, D)]],
    [D, D])

_combine1_call = _tc_call(
    _combine1_body,
    [_row_spec(D), _row_spec(D), _cnt_spec, _row_spec(D)] +
    [_full_spec(s) for s in
     [(D, D), (1, D), (D, D), (D, D), (1, D), (D, D), (1, D), (1, D)]],
    [D])

_combine2_call = _tc_call(
    _combine2_body,
    [_row_spec(D), _row_spec(D), _cnt_spec, _row_spec(D)] +
    [_full_spec(s) for s in
     [(D, D), (1, D), (D, D), (D, D), (1, D), (D, D),
      (D, D), (1, D), (D_OUT, D), (1, D_OUT)]],
    [D_OUT])


def kernel(x, edge_index_onset, edge_index_voice, Wp_on, bp_on, Wl1_on,
           bl1_on, Wr1_on, Wp_vo, bp_vo, Wl1_vo, bl1_vo, Wr1_vo, Wl2_on,
           bl2_on, Wr2_on, Wl2_vo, bl2_vo, Wr2_vo, ln_g, ln_b, W1, b1,
           W2, b2):
    zeros = jnp.zeros((N, D), jnp.float32)

    r2 = lambda v: v.reshape(1, -1)

    h_on, h_vo = _proj_call(x, Wp_on, r2(bp_on), Wp_vo, r2(bp_vo))
    e_on = edge_index_onset.reshape(2 * E)
    e_vo = edge_index_voice.reshape(2 * E)
    s_on, s_vo, cnt_p = _seg_call(True)(h_on, e_on, h_vo, e_vo, zeros)
    cnt = jnp.transpose(cnt_p.reshape(2 * NS, N), (1, 0))
    (h,) = _combine1_call(s_on, s_vo, cnt, x,
                          Wl1_on, r2(bl1_on), Wr1_on,
                          Wl1_vo, r2(bl1_vo), Wr1_vo,
                          r2(ln_g), r2(ln_b))
    s2_on, s2_vo = _seg_call(False)(h, e_on, h, e_vo, zeros)
    (out,) = _combine2_call(s2_on, s2_vo, cnt, h,
                            Wl2_on, r2(bl2_on), Wr2_on,
                            Wl2_vo, r2(bl2_vo), Wr2_vo,
                            W1, r2(b1), W2, r2(b2))
    return out
